# Initial kernel scaffold; baseline (speedup 1.0000x reference)
import jax
import jax.numpy as jnp
from jax.experimental import pallas as pl


def kernel(z, pos, edge_index, batch, atom_embed, W_msg, b_msg, W_upd, b_upd, W_out, b_out, vdw_radii):
    raise NotImplementedError("write your pallas kernel here")


# speedup vs baseline: 2.8600x; 2.8600x over previous
import functools

import numpy as np
import jax
import jax.numpy as jnp
from jax import lax
from jax.experimental import pallas as pl
from jax.experimental.pallas import tpu as pltpu
from jax.experimental.pallas import tpu_sc as plsc

N = 10000
E = 320000
D = 128
NG = 256
NTYPES = 100
TSTD = 0.0271
TMEAN = 0.6226

NT = 32
NCHUNK = 80
CK = 128
EPAD = NT * NCHUNK * CK
NPAD = 10240
RPT = NPAD // 16
ZR = 128

NBLK = 2000
WBLK = 4096

_mesh = plsc.VectorSubcoreMesh(core_axis_name="c", subcore_axis_name="s")
_f32 = jnp.float32


def _sc_edge_pos_body(pos16, srcg, dstg, d2g, src_v, dst_v, a_v, b_v):
    cid = lax.axis_index("c")
    sid = lax.axis_index("s")
    wid = cid * 16 + sid
    pltpu.sync_copy(srcg.at[wid], src_v)
    pltpu.sync_copy(dstg.at[wid], dst_v)

    @pl.loop(0, NCHUNK)
    def _chunk(c):
        pltpu.sync_copy(pos16.at[src_v.at[c]], a_v)
        pltpu.sync_copy(pos16.at[dst_v.at[c]], b_v)

        @pl.loop(0, CK)
        def _edge(k):
            dvec = a_v[k, :] - b_v[k, :]
            a_v[k, :] = dvec * dvec

        pltpu.sync_copy(a_v, d2g.at[wid, c])


_sc_edge_pos = functools.partial(
    pl.kernel,
    out_type=jax.ShapeDtypeStruct((NT, NCHUNK, CK, 16), _f32),
    mesh=_mesh,
    compiler_params=pltpu.CompilerParams(use_tc_tiling_on_sc=False),
    scratch_types=[
        pltpu.VMEM((NCHUNK, CK), jnp.int32),
        pltpu.VMEM((NCHUNK, CK), jnp.int32),
        pltpu.VMEM((CK, 16), _f32),
        pltpu.VMEM((CK, 16), _f32),
    ],
)(_sc_edge_pos_body)


def _sc_scatter_body(m_hbm, srcg, dstg, wg, agg_hbm,
                     agg_sh, src_v, dst_v, w_v, rows_v):
    cid = lax.axis_index("c")
    sid = lax.axis_index("s")
    wid = cid * 16 + sid

    @pl.loop(0, ZR)
    def _zrow(r):
        for j in range(8):
            rows_v[r, pl.ds(j * 16, 16)] = jnp.zeros((16,), _f32)

    for t in range(5):
        pltpu.sync_copy(rows_v, agg_sh.at[pl.ds(sid * RPT + t * ZR, ZR)])
    plsc.subcore_barrier()

    pltpu.sync_copy(srcg.at[wid], src_v)
    pltpu.sync_copy(dstg.at[wid], dst_v)
    pltpu.sync_copy(wg.at[wid], w_v)

    @pl.loop(0, NCHUNK)
    def _chunk(c):
        pltpu.sync_copy(m_hbm.at[src_v.at[c]], rows_v)

        @pl.loop(0, CK, step=16)
        def _e16(k0):
            wv = w_v[c, pl.ds(k0, 16)]
            for kk in range(16):
                wk = wv[kk]
                for j in range(8):
                    sl = (k0 + kk, pl.ds(j * 16, 16))
                    rows_v[sl] = rows_v[sl] * wk

        pltpu.sync_copy(rows_v, agg_sh.at[dst_v.at[c]], add=True)

    plsc.subcore_barrier()
    for t in range(5):
        pltpu.sync_copy(agg_sh.at[pl.ds(sid * RPT + t * ZR, ZR)], rows_v)
        pltpu.sync_copy(rows_v,
                        agg_hbm.at[cid, pl.ds(sid * RPT + t * ZR, ZR)])


_sc_scatter = functools.partial(
    pl.kernel,
    out_type=jax.ShapeDtypeStruct((2, NPAD, D), _f32),
    mesh=_mesh,
    scratch_types=[
        pltpu.VMEM_SHARED((NPAD, D), _f32),
        pltpu.VMEM((NCHUNK, CK), jnp.int32),
        pltpu.VMEM((NCHUNK, CK), jnp.int32),
        pltpu.VMEM((NCHUNK, CK), _f32),
        pltpu.VMEM((CK, D), _f32),
    ],
)(_sc_scatter_body)


def _tc_embed_body(z_ref, emb_ref, wm_ref, bm_ref, vdw_ref,
                   h_ref, m_ref, vol_ref):
    z = z_ref[...]
    oh = (z == lax.broadcasted_iota(jnp.int32, (NBLK, NTYPES), 1)
          ).astype(_f32)
    h = jnp.dot(oh, emb_ref[...], preferred_element_type=_f32)
    h_ref[...] = h
    m_ref[...] = jnp.maximum(
        jnp.dot(h, wm_ref[...], preferred_element_type=_f32) + bm_ref[...],
        0.0)
    r = vdw_ref[...]
    vol_ref[...] = jnp.dot(oh, (4.0 / 3.0) * np.pi * r * r * r,
                           preferred_element_type=_f32)


def _tc_embed(z2, atom_embed, W_msg, b_msg2, vdw2):
    return pl.pallas_call(
        _tc_embed_body,
        grid=(N // NBLK,),
        in_specs=[
            pl.BlockSpec((NBLK, 1), lambda i: (i, 0)),
            pl.BlockSpec((NTYPES, D), lambda i: (0, 0)),
            pl.BlockSpec((D, D), lambda i: (0, 0)),
            pl.BlockSpec((1, D), lambda i: (0, 0)),
            pl.BlockSpec((NTYPES, 1), lambda i: (0, 0)),
        ],
        out_specs=[
            pl.BlockSpec((NBLK, D), lambda i: (i, 0)),
            pl.BlockSpec((NBLK, D), lambda i: (i, 0)),
            pl.BlockSpec((NBLK, 1), lambda i: (i, 0)),
        ],
        out_shape=[
            jax.ShapeDtypeStruct((N, D), _f32),
            jax.ShapeDtypeStruct((N, D), _f32),
            jax.ShapeDtypeStruct((N, 1), _f32),
        ],
    )(z2, atom_embed, W_msg, b_msg2, vdw2)


def _tc_w_body(d2_ref, g_ref, w_ref):
    s = jnp.dot(d2_ref[...], g_ref[...], preferred_element_type=_f32)
    w = jnp.exp(-jnp.sqrt(s))
    i = pl.program_id(0)
    row = lax.broadcasted_iota(jnp.int32, s.shape, 0)
    col = lax.broadcasted_iota(jnp.int32, s.shape, 1)
    e = (i * WBLK + row) * 8 + col
    w_ref[...] = jnp.where(e < E, w, 0.0)


def _tc_w(d2m, gmat):
    return pl.pallas_call(
        _tc_w_body,
        grid=(EPAD // 8 // WBLK,),
        in_specs=[
            pl.BlockSpec((WBLK, 128), lambda i: (i, 0)),
            pl.BlockSpec((128, 8), lambda i: (0, 0)),
        ],
        out_specs=pl.BlockSpec((WBLK, 8), lambda i: (i, 0)),
        out_shape=jax.ShapeDtypeStruct((EPAD // 8, 8), _f32),
    )(d2m, gmat)


def _tc_round_body(agg_ref, h_ref, wu_ref, bu_ref, wm_ref, bm_ref,
                   hn_ref, mn_ref):
    a = agg_ref[0] + agg_ref[1]
    hn = jnp.maximum(
        jnp.dot(a, wu_ref[...], preferred_element_type=_f32)
        + bu_ref[...] + h_ref[...], 0.0)
    hn_ref[...] = hn
    if mn_ref is not None:
        mn_ref[...] = jnp.maximum(
            jnp.dot(hn, wm_ref[...], preferred_element_type=_f32)
            + bm_ref[...], 0.0)


def _tc_round(agg, h, W_upd, b_upd2, W_msg, b_msg2, last):
    body = (functools.partial(_tc_round_body, mn_ref=None) if last
            else _tc_round_body)
    out_specs = [pl.BlockSpec((NBLK, D), lambda i: (i, 0))]
    out_shape = [jax.ShapeDtypeStruct((N, D), _f32)]
    if not last:
        out_specs.append(pl.BlockSpec((NBLK, D), lambda i: (i, 0)))
        out_shape.append(jax.ShapeDtypeStruct((N, D), _f32))
    return pl.pallas_call(
        body,
        grid=(N // NBLK,),
        in_specs=[
            pl.BlockSpec((2, NBLK, D), lambda i: (0, i, 0)),
            pl.BlockSpec((NBLK, D), lambda i: (i, 0)),
            pl.BlockSpec((D, D), lambda i: (0, 0)),
            pl.BlockSpec((1, D), lambda i: (0, 0)),
            pl.BlockSpec((D, D), lambda i: (0, 0)),
            pl.BlockSpec((1, D), lambda i: (0, 0)),
        ],
        out_specs=out_specs,
        out_shape=out_shape,
    )(agg, h, W_upd, b_upd2, W_msg, b_msg2)


def _tc_pool_body(h_ref, batch_ref, vol_ref, wo_ref, bo_ref,
                  pred_ref, mvol_ref, g_sc, cnt_sc, vol_sc):
    i = pl.program_id(0)

    @pl.when(i == 0)
    def _init():
        g_sc[...] = jnp.zeros_like(g_sc)
        cnt_sc[...] = jnp.zeros_like(cnt_sc)
        vol_sc[...] = jnp.zeros_like(vol_sc)

    ohT = (lax.broadcasted_iota(jnp.int32, (NG, NBLK), 0) == batch_ref[0]
           ).astype(_f32)
    g_sc[...] += jnp.dot(ohT, h_ref[...], preferred_element_type=_f32)
    cnt_sc[...] += jnp.sum(ohT, axis=1, keepdims=True)
    vol_sc[...] += jnp.dot(ohT, vol_ref[...], preferred_element_type=_f32)

    @pl.when(i == N // NBLK - 1)
    def _fin():
        gm = g_sc[...] / jnp.maximum(cnt_sc[...], 1.0)
        pred = jnp.dot(gm, wo_ref[...], preferred_element_type=_f32) \
            + bo_ref[...]
        pred_ref[...] = pred * TSTD + TMEAN
        mvol_ref[...] = vol_sc[...]


def _tc_pool(h, batch2, vol, W_out, b_out2):
    return pl.pallas_call(
        _tc_pool_body,
        grid=(N // NBLK,),
        in_specs=[
            pl.BlockSpec((NBLK, D), lambda i: (i, 0)),
            pl.BlockSpec((1, 1, NBLK), lambda i: (i, 0, 0)),
            pl.BlockSpec((NBLK, 1), lambda i: (i, 0)),
            pl.BlockSpec((D, 1), lambda i: (0, 0)),
            pl.BlockSpec((1, 1), lambda i: (0, 0)),
        ],
        out_specs=[
            pl.BlockSpec((NG, 1), lambda i: (0, 0)),
            pl.BlockSpec((NG, 1), lambda i: (0, 0)),
        ],
        out_shape=[
            jax.ShapeDtypeStruct((NG, 1), _f32),
            jax.ShapeDtypeStruct((NG, 1), _f32),
        ],
        scratch_shapes=[
            pltpu.VMEM((NG, D), _f32),
            pltpu.VMEM((NG, 1), _f32),
            pltpu.VMEM((NG, 1), _f32),
        ],
    )(h, batch2, vol, W_out, b_out2)


_GMAT = np.kron(np.eye(8, dtype=np.float32), np.ones((16, 1), np.float32))


def kernel(z, pos, edge_index, batch, atom_embed, W_msg, b_msg, W_upd, b_upd,
           W_out, b_out, vdw_radii):
    src = edge_index[0].astype(jnp.int32)
    dst = edge_index[1].astype(jnp.int32)
    padn = EPAD - E
    zpad = jnp.zeros((padn,), jnp.int32)
    srcg = jnp.concatenate([src, zpad]).reshape(NT, NCHUNK, CK)
    dstg = jnp.concatenate([dst, zpad]).reshape(NT, NCHUNK, CK)
    pos16 = jnp.pad(pos.astype(_f32), ((0, 0), (0, 13)))
    z2 = z.astype(jnp.int32).reshape(N, 1)
    batch2 = batch.astype(jnp.int32).reshape(N // NBLK, 1, NBLK)
    b_msg2 = b_msg.reshape(1, D)
    b_upd2 = b_upd.reshape(1, D)
    b_out2 = b_out.reshape(1, 1)
    vdw2 = vdw_radii.reshape(NTYPES, 1)
    gmat = jnp.asarray(_GMAT)

    d2g = _sc_edge_pos(pos16, srcg, dstg)
    h, m, vol = _tc_embed(z2, atom_embed, W_msg, b_msg2, vdw2)
    wflat = _tc_w(d2g.reshape(EPAD // 8, 128), gmat)
    wg = wflat.reshape(NT, NCHUNK, CK)

    for r in range(3):
        agg = _sc_scatter(m, srcg, dstg, wg)
        outs = _tc_round(agg, h, W_upd, b_upd2, W_msg, b_msg2, last=(r == 2))
        if r < 2:
            h, m = outs
        else:
            (h,) = outs

    pred2, mvol2 = _tc_pool(h, batch2, vol, W_out, b_out2)
    return pred2.reshape(NG), mvol2.reshape(NG)


# speedup vs baseline: 3.7350x; 1.3060x over previous
import functools

import numpy as np
import jax
import jax.numpy as jnp
from jax import lax
from jax.experimental import pallas as pl
from jax.experimental.pallas import tpu as pltpu
from jax.experimental.pallas import tpu_sc as plsc

N = 10000
E = 320000
D = 128
NG = 256
NTYPES = 100
TSTD = 0.0271
TMEAN = 0.6226

NT = 32
NCHUNK = 80
CK = 128
NCS = 160
CKS = 64
EPAD = NT * NCHUNK * CK
NPAD = 10240
RPT = NPAD // 16
ZR = 128

NBLK = 2000
WBLK = 4096

_mesh = plsc.VectorSubcoreMesh(core_axis_name="c", subcore_axis_name="s")
_f32 = jnp.float32


def _sc_edge_pos_body(pos16, srcg, dstg, d2g, src_v, dst_v, a_v, b_v):
    cid = lax.axis_index("c")
    sid = lax.axis_index("s")
    wid = cid * 16 + sid
    pltpu.sync_copy(srcg.at[wid], src_v)
    pltpu.sync_copy(dstg.at[wid], dst_v)

    @pl.loop(0, NCHUNK)
    def _chunk(c):
        pltpu.sync_copy(pos16.at[src_v.at[c]], a_v)
        pltpu.sync_copy(pos16.at[dst_v.at[c]], b_v)

        @pl.loop(0, CK)
        def _edge(k):
            dvec = a_v[k, :] - b_v[k, :]
            a_v[k, :] = dvec * dvec

        pltpu.sync_copy(a_v, d2g.at[wid, c])


_sc_edge_pos = functools.partial(
    pl.kernel,
    out_type=jax.ShapeDtypeStruct((NT, NCHUNK, CK, 16), _f32),
    mesh=_mesh,
    compiler_params=pltpu.CompilerParams(use_tc_tiling_on_sc=False),
    scratch_types=[
        pltpu.VMEM((NCHUNK, CK), jnp.int32),
        pltpu.VMEM((NCHUNK, CK), jnp.int32),
        pltpu.VMEM((CK, 16), _f32),
        pltpu.VMEM((CK, 16), _f32),
    ],
)(_sc_edge_pos_body)


def _sc_scatter_body(m_hbm, sd_hbm, wg_hbm, agg_hbm,
                     agg_sh, w_v, rows, ebuf, gsem, ssem, esem):
    cid = lax.axis_index("c")
    sid = lax.axis_index("s")
    wid = cid * 16 + sid

    @pl.loop(0, CKS)
    def _zrow(r):
        for j in range(8):
            rows[0, r, pl.ds(j * 16, 16)] = jnp.zeros((16,), _f32)

    for t in range(RPT // CKS):
        pltpu.sync_copy(rows.at[0],
                        agg_sh.at[pl.ds(sid * RPT + t * CKS, CKS)])
    plsc.subcore_barrier()

    pltpu.sync_copy(wg_hbm.at[wid], w_v)
    for e in range(4):
        pltpu.async_copy(sd_hbm.at[wid, e], ebuf.at[e], esem.at[e])
    for s in range(2):
        pltpu.make_async_copy(sd_hbm.at[wid, s], ebuf.at[s],
                              esem.at[s]).wait()
        pltpu.async_copy(m_hbm.at[ebuf.at[s, 0]], rows.at[s], gsem.at[s])

    @pl.loop(0, NCS, step=8)
    def _grp(c0):
        for off in range(8):
            cc = c0 + off
            s = off % 4
            e = off
            s2 = (off + 2) % 4
            e2 = (off + 2) % 8
            e4 = (off + 4) % 8
            em2 = (off + 6) % 8

            pltpu.make_async_copy(m_hbm.at[ebuf.at[e, 0]], rows.at[s],
                                  gsem.at[s]).wait()

            @pl.when(cc + 4 < NCS)
            def _pf():
                pltpu.async_copy(sd_hbm.at[wid, cc + 4], ebuf.at[e4],
                                 esem.at[e4])

            @pl.when(cc + 2 < NCS)
            def _gnext():
                @pl.when(cc >= 2)
                def _wsc():
                    pltpu.make_async_copy(
                        rows.at[s2], agg_sh.at[ebuf.at[em2, 1]],
                        ssem.at[s2]).wait()
                pltpu.make_async_copy(sd_hbm.at[wid, cc + 2], ebuf.at[e2],
                                      esem.at[e2]).wait()
                pltpu.async_copy(m_hbm.at[ebuf.at[e2, 0]], rows.at[s2],
                                 gsem.at[s2])

            @pl.loop(0, CKS, step=16)
            def _mul(k0):
                wv = w_v[cc // 2, pl.ds((cc % 2) * CKS + k0, 16)]
                for kk in range(16):
                    wk = wv[kk]
                    for j in range(8):
                        sl = (s, k0 + kk, pl.ds(j * 16, 16))
                        rows[sl] = rows[sl] * wk

            pltpu.async_copy(rows.at[s], agg_sh.at[ebuf.at[e, 1]],
                             ssem.at[s], add=True)

    for cc in range(NCS - 4, NCS):
        s = cc % 4
        e = cc % 8
        pltpu.make_async_copy(rows.at[s], agg_sh.at[ebuf.at[e, 1]],
                              ssem.at[s]).wait()

    plsc.subcore_barrier()
    for t in range(RPT // CKS):
        pltpu.sync_copy(agg_sh.at[pl.ds(sid * RPT + t * CKS, CKS)],
                        rows.at[0])
        pltpu.sync_copy(rows.at[0],
                        agg_hbm.at[cid, pl.ds(sid * RPT + t * CKS, CKS)])


_sc_scatter = functools.partial(
    pl.kernel,
    out_type=jax.ShapeDtypeStruct((2, NPAD, D), _f32),
    mesh=_mesh,
    scratch_types=[
        pltpu.VMEM_SHARED((NPAD, D), _f32),
        pltpu.VMEM((NCS * CKS // 128, 128), _f32),
        pltpu.VMEM((4, CKS, D), _f32),
        pltpu.VMEM((8, 2, CKS), jnp.int32),
        pltpu.SemaphoreType.DMA((4,)),
        pltpu.SemaphoreType.DMA((4,)),
        pltpu.SemaphoreType.DMA((8,)),
    ],
)(_sc_scatter_body)


def _tc_embed_body(z_ref, emb_ref, wm_ref, bm_ref, vdw_ref,
                   h_ref, m_ref, vol_ref):
    z = z_ref[...]
    oh = (z == lax.broadcasted_iota(jnp.int32, (NBLK, NTYPES), 1)
          ).astype(_f32)
    h = jnp.dot(oh, emb_ref[...], preferred_element_type=_f32)
    h_ref[...] = h
    m_ref[...] = jnp.maximum(
        jnp.dot(h, wm_ref[...], preferred_element_type=_f32) + bm_ref[...],
        0.0)
    r = vdw_ref[...]
    vol_ref[...] = jnp.dot(oh, (4.0 / 3.0) * np.pi * r * r * r,
                           preferred_element_type=_f32)


def _tc_embed(z2, atom_embed, W_msg, b_msg2, vdw2):
    return pl.pallas_call(
        _tc_embed_body,
        grid=(N // NBLK,),
        in_specs=[
            pl.BlockSpec((NBLK, 1), lambda i: (i, 0)),
            pl.BlockSpec((NTYPES, D), lambda i: (0, 0)),
            pl.BlockSpec((D, D), lambda i: (0, 0)),
            pl.BlockSpec((1, D), lambda i: (0, 0)),
            pl.BlockSpec((NTYPES, 1), lambda i: (0, 0)),
        ],
        out_specs=[
            pl.BlockSpec((NBLK, D), lambda i: (i, 0)),
            pl.BlockSpec((NBLK, D), lambda i: (i, 0)),
            pl.BlockSpec((NBLK, 1), lambda i: (i, 0)),
        ],
        out_shape=[
            jax.ShapeDtypeStruct((N, D), _f32),
            jax.ShapeDtypeStruct((N, D), _f32),
            jax.ShapeDtypeStruct((N, 1), _f32),
        ],
    )(z2, atom_embed, W_msg, b_msg2, vdw2)


def _tc_w_body(d2_ref, g_ref, w_ref):
    s = jnp.dot(d2_ref[...], g_ref[...], preferred_element_type=_f32)
    w = jnp.exp(-jnp.sqrt(s))
    i = pl.program_id(0)
    row = lax.broadcasted_iota(jnp.int32, s.shape, 0)
    col = lax.broadcasted_iota(jnp.int32, s.shape, 1)
    e = (i * WBLK + row) * 8 + col
    w_ref[...] = jnp.where(e < E, w, 0.0)


def _tc_w(d2m, gmat):
    return pl.pallas_call(
        _tc_w_body,
        grid=(EPAD // 8 // WBLK,),
        in_specs=[
            pl.BlockSpec((WBLK, 128), lambda i: (i, 0)),
            pl.BlockSpec((128, 8), lambda i: (0, 0)),
        ],
        out_specs=pl.BlockSpec((WBLK, 8), lambda i: (i, 0)),
        out_shape=jax.ShapeDtypeStruct((EPAD // 8, 8), _f32),
    )(d2m, gmat)


def _tc_round_body(agg_ref, h_ref, wu_ref, bu_ref, wm_ref, bm_ref,
                   hn_ref, mn_ref):
    a = agg_ref[0] + agg_ref[1]
    hn = jnp.maximum(
        jnp.dot(a, wu_ref[...], preferred_element_type=_f32)
        + bu_ref[...] + h_ref[...], 0.0)
    hn_ref[...] = hn
    if mn_ref is not None:
        mn_ref[...] = jnp.maximum(
            jnp.dot(hn, wm_ref[...], preferred_element_type=_f32)
            + bm_ref[...], 0.0)


def _tc_round(agg, h, W_upd, b_upd2, W_msg, b_msg2, last):
    body = (functools.partial(_tc_round_body, mn_ref=None) if last
            else _tc_round_body)
    out_specs = [pl.BlockSpec((NBLK, D), lambda i: (i, 0))]
    out_shape = [jax.ShapeDtypeStruct((N, D), _f32)]
    if not last:
        out_specs.append(pl.BlockSpec((NBLK, D), lambda i: (i, 0)))
        out_shape.append(jax.ShapeDtypeStruct((N, D), _f32))
    return pl.pallas_call(
        body,
        grid=(N // NBLK,),
        in_specs=[
            pl.BlockSpec((2, NBLK, D), lambda i: (0, i, 0)),
            pl.BlockSpec((NBLK, D), lambda i: (i, 0)),
            pl.BlockSpec((D, D), lambda i: (0, 0)),
            pl.BlockSpec((1, D), lambda i: (0, 0)),
            pl.BlockSpec((D, D), lambda i: (0, 0)),
            pl.BlockSpec((1, D), lambda i: (0, 0)),
        ],
        out_specs=out_specs,
        out_shape=out_shape,
    )(agg, h, W_upd, b_upd2, W_msg, b_msg2)


def _tc_pool_body(h_ref, batch_ref, vol_ref, wo_ref, bo_ref,
                  pred_ref, mvol_ref, g_sc, cnt_sc, vol_sc):
    i = pl.program_id(0)

    @pl.when(i == 0)
    def _init():
        g_sc[...] = jnp.zeros_like(g_sc)
        cnt_sc[...] = jnp.zeros_like(cnt_sc)
        vol_sc[...] = jnp.zeros_like(vol_sc)

    ohT = (lax.broadcasted_iota(jnp.int32, (NG, NBLK), 0) == batch_ref[0]
           ).astype(_f32)
    g_sc[...] += jnp.dot(ohT, h_ref[...], preferred_element_type=_f32)
    cnt_sc[...] += jnp.sum(ohT, axis=1, keepdims=True)
    vol_sc[...] += jnp.dot(ohT, vol_ref[...], preferred_element_type=_f32)

    @pl.when(i == N // NBLK - 1)
    def _fin():
        gm = g_sc[...] / jnp.maximum(cnt_sc[...], 1.0)
        pred = jnp.dot(gm, wo_ref[...], preferred_element_type=_f32) \
            + bo_ref[...]
        pred_ref[...] = pred * TSTD + TMEAN
        mvol_ref[...] = vol_sc[...]


def _tc_pool(h, batch2, vol, W_out, b_out2):
    return pl.pallas_call(
        _tc_pool_body,
        grid=(N // NBLK,),
        in_specs=[
            pl.BlockSpec((NBLK, D), lambda i: (i, 0)),
            pl.BlockSpec((1, 1, NBLK), lambda i: (i, 0, 0)),
            pl.BlockSpec((NBLK, 1), lambda i: (i, 0)),
            pl.BlockSpec((D, 1), lambda i: (0, 0)),
            pl.BlockSpec((1, 1), lambda i: (0, 0)),
        ],
        out_specs=[
            pl.BlockSpec((NG, 1), lambda i: (0, 0)),
            pl.BlockSpec((NG, 1), lambda i: (0, 0)),
        ],
        out_shape=[
            jax.ShapeDtypeStruct((NG, 1), _f32),
            jax.ShapeDtypeStruct((NG, 1), _f32),
        ],
        scratch_shapes=[
            pltpu.VMEM((NG, D), _f32),
            pltpu.VMEM((NG, 1), _f32),
            pltpu.VMEM((NG, 1), _f32),
        ],
    )(h, batch2, vol, W_out, b_out2)


_GMAT = np.kron(np.eye(8, dtype=np.float32), np.ones((16, 1), np.float32))


def kernel(z, pos, edge_index, batch, atom_embed, W_msg, b_msg, W_upd, b_upd,
           W_out, b_out, vdw_radii):
    src = edge_index[0].astype(jnp.int32)
    dst = edge_index[1].astype(jnp.int32)
    padn = EPAD - E
    zpad = jnp.zeros((padn,), jnp.int32)
    srcp = jnp.concatenate([src, zpad])
    dstp = jnp.concatenate([dst, zpad])
    srcg = srcp.reshape(NT, NCHUNK, CK)
    dstg = dstp.reshape(NT, NCHUNK, CK)
    sdg = jnp.concatenate([srcp.reshape(NT, NCS, 1, CKS),
                           dstp.reshape(NT, NCS, 1, CKS)], axis=2)
    pos16 = jnp.pad(pos.astype(_f32), ((0, 0), (0, 13)))
    z2 = z.astype(jnp.int32).reshape(N, 1)
    batch2 = batch.astype(jnp.int32).reshape(N // NBLK, 1, NBLK)
    b_msg2 = b_msg.reshape(1, D)
    b_upd2 = b_upd.reshape(1, D)
    b_out2 = b_out.reshape(1, 1)
    vdw2 = vdw_radii.reshape(NTYPES, 1)
    gmat = jnp.asarray(_GMAT)

    d2g = _sc_edge_pos(pos16, srcg, dstg)
    h, m, vol = _tc_embed(z2, atom_embed, W_msg, b_msg2, vdw2)
    wflat = _tc_w(d2g.reshape(EPAD // 8, 128), gmat)
    wgs = wflat.reshape(NT, NCS * CKS // 128, 128)

    for r in range(3):
        agg = _sc_scatter(m, sdg, wgs)
        outs = _tc_round(agg, h, W_upd, b_upd2, W_msg, b_msg2, last=(r == 2))
        if r < 2:
            h, m = outs
        else:
            (h,) = outs

    pred2, mvol2 = _tc_pool(h, batch2, vol, W_out, b_out2)
    return pred2.reshape(NG), mvol2.reshape(NG)
